# Initial kernel scaffold; baseline (speedup 1.0000x reference)
import jax
import jax.numpy as jnp
from jax.experimental import pallas as pl


def kernel(inputs, embedding):
    raise NotImplementedError("write your pallas kernel here")


# speedup vs baseline: 2.5505x; 2.5505x over previous
import jax
import jax.numpy as jnp
from jax.experimental import pallas as pl

_NUM_EMB = 512
_DIM = 32
_COMMIT = 0.25
_TBLK = 2048


def _vq_block_kernel(x_ref, emb_ref, q_ref, idx_ref, loss_ref):
    x = x_ref[0]
    emb = emb_ref[...]
    flat = x.T
    flat_sq = jnp.sum(flat * flat, axis=1, keepdims=True)
    e_sq = jnp.sum(emb * emb, axis=1)
    cross = jnp.dot(flat, emb.T, preferred_element_type=jnp.float32)
    dists = flat_sq + e_sq[None, :] - 2.0 * cross
    m = jnp.min(dists, axis=1, keepdims=True)
    lane = jax.lax.broadcasted_iota(jnp.int32, dists.shape, 1)
    idx = jnp.min(jnp.where(dists == m, lane, _NUM_EMB), axis=1)
    onehot = (lane == idx[:, None]).astype(jnp.float32)
    qflat = jnp.dot(onehot, emb, preferred_element_type=jnp.float32)
    diff = qflat - flat
    q_ref[0] = (flat + diff).T
    idx_ref[0, 0, 0] = idx
    loss_ref[...] = jnp.sum(diff * diff).reshape(1, 1, 1, 1)


def kernel(inputs, embedding):
    B, D, T = inputs.shape
    nt = T // _TBLK
    q_st, idx4, partials = pl.pallas_call(
        _vq_block_kernel,
        grid=(B, nt),
        in_specs=[
            pl.BlockSpec((1, D, _TBLK), lambda b, t: (b, 0, t)),
            pl.BlockSpec((_NUM_EMB, D), lambda b, t: (0, 0)),
        ],
        out_specs=[
            pl.BlockSpec((1, D, _TBLK), lambda b, t: (b, 0, t)),
            pl.BlockSpec((1, 1, 1, _TBLK), lambda b, t: (b, t, 0, 0)),
            pl.BlockSpec((1, 1, 1, 1), lambda b, t: (b, t, 0, 0)),
        ],
        out_shape=[
            jax.ShapeDtypeStruct((B, D, T), jnp.float32),
            jax.ShapeDtypeStruct((B, nt, 1, _TBLK), jnp.int32),
            jax.ShapeDtypeStruct((B, nt, 1, 1), jnp.float32),
        ],
    )(inputs, embedding)
    indices = idx4.reshape(B, T)
    mse = jnp.sum(partials) / (B * D * T)
    loss_vq = mse + _COMMIT * mse
    return (q_st, loss_vq, indices)


# speedup vs baseline: 4.7373x; 1.8574x over previous
import jax
import jax.numpy as jnp
from jax import lax
from jax.experimental import pallas as pl

_NUM_EMB = 512
_DIM = 32
_COMMIT = 0.25
_TBLK = 2048


def _vq_block_kernel(x_ref, emb_ref, q_ref, idx_ref, loss_ref):
    x = x_ref[0]
    emb = emb_ref[...]
    flat_sq = jnp.sum(x * x, axis=0, keepdims=True)
    e_sq = jnp.sum(emb * emb, axis=1, keepdims=True)
    cross2 = jnp.dot(emb + emb, x, preferred_element_type=jnp.float32)
    dists = (flat_sq + e_sq) - cross2
    m = jnp.min(dists, axis=0, keepdims=True)
    code = jax.lax.broadcasted_iota(jnp.int32, dists.shape, 0)
    sel = jnp.where(dists == m, code, _NUM_EMB)
    idx = jnp.min(sel, axis=0, keepdims=True)
    onehot = (code == idx).astype(jnp.float32)
    q = lax.dot_general(emb, onehot, (((0,), (0,)), ((), ())),
                        preferred_element_type=jnp.float32)
    diff = q - x
    q_ref[0] = x + diff
    idx_ref[0, 0, 0] = idx[0]
    loss_ref[...] = jnp.sum(diff * diff).reshape(1, 1, 1, 1)


def kernel(inputs, embedding):
    B, D, T = inputs.shape
    nt = T // _TBLK
    q_st, idx3, partials = pl.pallas_call(
        _vq_block_kernel,
        grid=(B, nt),
        in_specs=[
            pl.BlockSpec((1, D, _TBLK), lambda b, t: (b, 0, t)),
            pl.BlockSpec((_NUM_EMB, D), lambda b, t: (0, 0)),
        ],
        out_specs=[
            pl.BlockSpec((1, D, _TBLK), lambda b, t: (b, 0, t)),
            pl.BlockSpec((1, 1, 1, _TBLK), lambda b, t: (b, t, 0, 0)),
            pl.BlockSpec((1, 1, 1, 1), lambda b, t: (b, t, 0, 0)),
        ],
        out_shape=[
            jax.ShapeDtypeStruct((B, D, T), jnp.float32),
            jax.ShapeDtypeStruct((B, nt, 1, _TBLK), jnp.int32),
            jax.ShapeDtypeStruct((B, nt, 1, 1), jnp.float32),
        ],
    )(inputs, embedding)
    indices = idx3.reshape(B, T)
    mse = jnp.sum(partials) / (B * D * T)
    loss_vq = mse + _COMMIT * mse
    return (q_st, loss_vq, indices)


# speedup vs baseline: 6.6602x; 1.4059x over previous
import jax
import jax.numpy as jnp
from jax import lax
from jax.experimental import pallas as pl

_NUM_EMB = 512
_DIM = 32
_COMMIT = 0.25
_TBLK = 8192


def _vq_block_kernel(x_ref, emb_ref, q_ref, idx_ref, loss_ref):
    x = x_ref[0]
    emb = emb_ref[...]
    flat_sq = jnp.sum(x * x, axis=0, keepdims=True)
    e_sq = jnp.sum(emb * emb, axis=1, keepdims=True)
    cross2 = jnp.dot(emb + emb, x, preferred_element_type=jnp.float32)
    dists = (flat_sq + e_sq) - cross2
    m8 = dists[0:8]
    r8 = jnp.zeros((8, _TBLK), jnp.int32)
    for r in range(1, _NUM_EMB // 8):
        d_r = dists[8 * r:8 * r + 8]
        lt = d_r < m8
        m8 = jnp.where(lt, d_r, m8)
        r8 = jnp.where(lt, r, r8)
    sub8 = jax.lax.broadcasted_iota(jnp.int32, (8, _TBLK), 0)
    cand = r8 * 8 + sub8
    mf = jnp.min(m8, axis=0, keepdims=True)
    idx = jnp.min(jnp.where(m8 == mf, cand, _NUM_EMB), axis=0, keepdims=True)
    code = jax.lax.broadcasted_iota(jnp.int32, dists.shape, 0)
    onehot = (code == idx).astype(jnp.float32)
    q = lax.dot_general(emb, onehot, (((0,), (0,)), ((), ())),
                        preferred_element_type=jnp.float32)
    diff = q - x
    q_ref[0] = x + diff
    idx_ref[0, 0, 0] = idx[0]
    loss_ref[...] = jnp.sum(diff * diff).reshape(1, 1, 1, 1)


def kernel(inputs, embedding):
    B, D, T = inputs.shape
    nt = T // _TBLK
    q_st, idx3, partials = pl.pallas_call(
        _vq_block_kernel,
        grid=(B, nt),
        in_specs=[
            pl.BlockSpec((1, D, _TBLK), lambda b, t: (b, 0, t)),
            pl.BlockSpec((_NUM_EMB, D), lambda b, t: (0, 0)),
        ],
        out_specs=[
            pl.BlockSpec((1, D, _TBLK), lambda b, t: (b, 0, t)),
            pl.BlockSpec((1, 1, 1, _TBLK), lambda b, t: (b, t, 0, 0)),
            pl.BlockSpec((1, 1, 1, 1), lambda b, t: (b, t, 0, 0)),
        ],
        out_shape=[
            jax.ShapeDtypeStruct((B, D, T), jnp.float32),
            jax.ShapeDtypeStruct((B, nt, 1, _TBLK), jnp.int32),
            jax.ShapeDtypeStruct((B, nt, 1, 1), jnp.float32),
        ],
    )(inputs, embedding)
    indices = idx3.reshape(B, T)
    mse = jnp.sum(partials) / (B * D * T)
    loss_vq = mse + _COMMIT * mse
    return (q_st, loss_vq, indices)
